# baseline (speedup 1.0000x reference)
import functools
import math

import jax
import jax.numpy as jnp
from jax import lax
from jax.experimental import pallas as pl
from jax.experimental.pallas import tpu as pltpu
from jax.experimental.pallas import tpu_sc as plsc

_SIZE = 8192
_PAD = 0
_SMOOTHING = 0.1
_CONF = 1.0 - _SMOOTHING
_EPS = _SMOOTHING / (_SIZE - 2)
_C = (_SIZE - 2) * _EPS * math.log(_EPS) + _CONF * math.log(_CONF)

_ROWS_PER_BLOCK = 256

_NC, _NS, _L = 2, 16, 16
_NW = _NC * _NS
_SEGS_PER_ROW = _SIZE // _L


def _tc_dense_block(x_ref, t_ref, out_ref):
    i = pl.program_id(0)

    @pl.when(i == 0)
    def _():
        out_ref[0, 0] = 0.0

    xb = x_ref[...]
    t = t_ref[0]
    rowsum = jnp.sum(xb, axis=1, keepdims=True)
    x0 = xb[:, 0:1]
    contrib = jnp.where(t != _PAD, _C - _EPS * rowsum + _EPS * x0, 0.0)
    out_ref[0, 0] += jnp.sum(contrib)


def _sc_gather_body(x_ref, t_ref, out_ref, t_v, idx_v, val_v, res_v, sem):
    wid = lax.axis_index("s") * _NC + lax.axis_index("c")
    rows_per_w = t_ref.shape[0] // _NW
    base = wid * rows_per_w
    pltpu.sync_copy(t_ref.at[pl.ds(base, rows_per_w)], t_v)

    lanes = lax.iota(jnp.int32, _L)
    acc = jnp.zeros((_L,), jnp.float32)
    n_chunks = rows_per_w // 128
    for k in range(n_chunks):
        for j in range(128 // _L):
            tc = t_v[pl.ds(k * 128 + j * _L, _L)]
            rows16 = base + k * 128 + j * _L + lanes
            idx_v[pl.ds(j * _L, _L)] = rows16 * _SIZE + tc
        pltpu.async_copy(x_ref.at[idx_v], val_v, sem).wait()
        for j in range(128 // _L):
            tc = t_v[pl.ds(k * 128 + j * _L, _L)]
            val = val_v[pl.ds(j * _L, _L)]
            acc = acc + jnp.where(tc != _PAD, val, 0.0)

    res_v[...] = (_EPS - _CONF) * acc
    pltpu.sync_copy(res_v, out_ref.at[wid])


def _sc_gather(x_flat, t_flat):
    body = functools.partial(
        pl.kernel,
        out_type=jax.ShapeDtypeStruct((_NW, _L), jnp.float32),
        mesh=plsc.VectorSubcoreMesh(core_axis_name="c", subcore_axis_name="s"),
        scratch_types=[
            pltpu.VMEM((256,), jnp.int32),
            pltpu.VMEM((128,), jnp.int32),
            pltpu.VMEM((128,), jnp.float32),
            pltpu.VMEM((_L,), jnp.float32),
            pltpu.SemaphoreType.DMA,
        ],
    )(_sc_gather_body)
    return body(x_flat, t_flat)


def kernel(x, target):
    B, S, V = x.shape
    rows = B * S
    rb = _ROWS_PER_BLOCK
    nblk = rows // rb
    x2 = x.reshape(rows, V)
    t_flat = target.reshape(rows).astype(jnp.int32)
    t3 = t_flat.reshape(nblk, rb, 1)

    xt_partials = _sc_gather(x2.reshape(rows * V), t_flat)

    dense = pl.pallas_call(
        _tc_dense_block,
        grid=(nblk,),
        in_specs=[
            pl.BlockSpec((rb, V), lambda i: (i, 0)),
            pl.BlockSpec((1, rb, 1), lambda i: (i, 0, 0)),
        ],
        out_specs=pl.BlockSpec(
            (1, 1), lambda i: (0, 0), memory_space=pltpu.SMEM
        ),
        out_shape=jax.ShapeDtypeStruct((1, 1), jnp.float32),
    )(x2, t3)

    return dense[0, 0] + jnp.sum(xt_partials)


# speedup vs baseline: 2.6524x; 2.6524x over previous
import functools
import math

import jax
import jax.numpy as jnp
from jax import lax
from jax.experimental import pallas as pl
from jax.experimental.pallas import tpu as pltpu
from jax.experimental.pallas import tpu_sc as plsc

_SIZE = 8192
_PAD = 0
_SMOOTHING = 0.1
_CONF = 1.0 - _SMOOTHING
_EPS = _SMOOTHING / (_SIZE - 2)
_C = (_SIZE - 2) * _EPS * math.log(_EPS) + _CONF * math.log(_CONF)

_ROWS_PER_BLOCK = 256

_NC, _NS, _L = 2, 16, 16
_NW = _NC * _NS
_SC_ROWS = 512
_W_ROWS = _SC_ROWS // _NW


def _tc_dense_block(x_ref, t_ref, out_ref):
    i = pl.program_id(0)

    @pl.when(i == 0)
    def _():
        out_ref[0, 0] = 0.0

    xb = x_ref[...]
    t = t_ref[0]
    rowsum = jnp.sum(xb, axis=1, keepdims=True)
    x0 = xb[:, 0:1]
    rb, v = xb.shape
    vocab_ids = lax.broadcasted_iota(jnp.int32, (rb, v), 1)
    xt = jnp.sum(jnp.where(vocab_ids == t, xb, 0.0), axis=1, keepdims=True)
    contrib = jnp.where(
        t != _PAD,
        _C - _EPS * rowsum + _EPS * x0 - (_CONF - _EPS) * xt,
        0.0,
    )
    out_ref[0, 0] += jnp.sum(contrib)


def _sc_body(x_ref, tb_ref, out_ref, tb_v, buf_v, res_v, sem):
    wid = lax.axis_index("s") * _NC + lax.axis_index("c")
    rows = tb_ref.shape[0] // _L
    r0 = rows - _SC_ROWS + wid * _W_ROWS
    lanes = lax.iota(jnp.int32, _L)
    out_acc = jnp.zeros((_L,), jnp.float32)

    for blk in range(_W_ROWS // 8):
        pltpu.async_copy(x_ref.at[pl.ds(r0 + blk * 8, 8)], buf_v, sem).wait()
        pltpu.sync_copy(
            tb_ref.at[pl.ds((r0 + blk * 8) * _L, 8 * _L)], tb_v
        )
        t_bs = [tb_v[pl.ds(s * _L, _L)] for s in range(8)]
        x0cs = [buf_v[s, pl.ds(0, _L)] for s in range(8)]

        def body(c2, carry, _t_bs=t_bs):
            accs = list(carry)
            base_c = c2 * 128
            for s in range(8):
                for c in range(8):
                    off = base_c + c * _L
                    chunk = buf_v[s, pl.ds(off, _L)]
                    accs[s] = accs[s] + chunk
                    v_ids = off + lanes
                    accs[8 + s] = accs[8 + s] + jnp.where(
                        v_ids == _t_bs[s], chunk, 0.0
                    )
            return tuple(accs)

        init = tuple(jnp.zeros((_L,), jnp.float32) for _ in range(16))
        accs = lax.fori_loop(0, _SIZE // 128, body, init)

        for s in range(8):
            contrib = (
                -_EPS * accs[s]
                - (_CONF - _EPS) * accs[8 + s]
                + jnp.where(lanes == 0, _C + _EPS * x0cs[s], 0.0)
            )
            out_acc = out_acc + jnp.where(t_bs[s] != _PAD, contrib, 0.0)

    res_v[...] = out_acc
    pltpu.sync_copy(res_v, out_ref.at[wid])


def _sc_tail(x2, t_bcast):
    body = functools.partial(
        pl.kernel,
        out_type=jax.ShapeDtypeStruct((_NW, _L), jnp.float32),
        mesh=plsc.VectorSubcoreMesh(core_axis_name="c", subcore_axis_name="s"),
        scratch_types=[
            pltpu.VMEM((8 * _L,), jnp.int32),
            pltpu.VMEM((8, _SIZE), jnp.float32),
            pltpu.VMEM((_L,), jnp.float32),
            pltpu.SemaphoreType.DMA,
        ],
    )(_sc_body)
    return body(x2, t_bcast)


def kernel(x, target):
    B, S, V = x.shape
    rows = B * S
    rb = _ROWS_PER_BLOCK
    tc_rows = rows - _SC_ROWS
    nblk = tc_rows // rb
    x2 = x.reshape(rows, V)
    t_flat = target.reshape(rows).astype(jnp.int32)
    t3 = t_flat[:tc_rows].reshape(nblk, rb, 1)

    t_bcast = jnp.broadcast_to(t_flat[:, None], (rows, _L)).reshape(rows * _L)
    sc_partials = _sc_tail(x2, t_bcast)

    dense = pl.pallas_call(
        _tc_dense_block,
        grid=(nblk,),
        in_specs=[
            pl.BlockSpec((rb, V), lambda i: (i, 0)),
            pl.BlockSpec((1, rb, 1), lambda i: (i, 0, 0)),
        ],
        out_specs=pl.BlockSpec(
            (1, 1), lambda i: (0, 0), memory_space=pltpu.SMEM
        ),
        out_shape=jax.ShapeDtypeStruct((1, 1), jnp.float32),
    )(x2, t3)

    return dense[0, 0] + jnp.sum(sc_partials)
